# Initial kernel scaffold; baseline (speedup 1.0000x reference)
import jax
import jax.numpy as jnp
from jax.experimental import pallas as pl


def kernel(ingredients, user, recipe, user_table, ing_table, user_bias, recipe_bias):
    raise NotImplementedError("write your pallas kernel here")


# speedup vs baseline: 2.0399x; 2.0399x over previous
import jax
import jax.numpy as jnp
from jax import lax
from jax.experimental import pallas as pl
from jax.experimental.pallas import tpu as pltpu
from jax.experimental.pallas import tpu_sc as plsc

B = 16384
L = 20
D = 64

NC = 2
NS = 16
NW = NC * NS
B_PER_W = B // NW
CH = 32
N_CHUNKS = B_PER_W // CH
IDX_ROWS = CH * L // 128


def _sc_body(ing_flat, user_h, recipe_h, ing_table, user_table, user_bias,
             recipe_bias, out_h, idx_v, rows_v, uidx_v, ridx_v, urows_v,
             ub_v, rb_v, out_v, m_v, sem, sem2):
    wid = lax.axis_index("s") * NC + lax.axis_index("c")
    lane = lax.iota(jnp.int32, 16)

    def chunk_body(c, _):
        base = wid * B_PER_W + c * CH

        pltpu.sync_copy(ing_flat.at[pl.ds(base * L, CH * L)], idx_v)
        pltpu.sync_copy(user_h.at[pl.ds(base, CH)], uidx_v)
        pltpu.sync_copy(recipe_h.at[pl.ds(base, CH)], ridx_v)

        cps = []
        for j in range(IDX_ROWS):
            cps.append(pltpu.async_copy(
                ing_table.at[idx_v.at[pl.ds(j * 128, 128)]],
                rows_v.at[pl.ds(j * 128, 128)], sem))
        u_cp = pltpu.async_copy(user_table.at[uidx_v], urows_v, sem2)
        ub_cp = pltpu.async_copy(user_bias.at[uidx_v], ub_v, sem2)
        rb_cp = pltpu.async_copy(recipe_bias.at[ridx_v], rb_v, sem2)
        for cp in cps:
            cp.wait()
        u_cp.wait()
        ub_cp.wait()
        rb_cp.wait()

        for g in range(CH // 16):
            def b_body(jj, carry, g=g):
                b = g * 16 + jj
                r = b * L
                v = jnp.zeros((16,), jnp.float32)
                for d in range(D // 16):
                    s = rows_v[r, pl.ds(d * 16, 16)]
                    for l in range(1, L):
                        s = s + rows_v[r + l, pl.ds(d * 16, 16)]
                    v = v + s * urows_v[b, pl.ds(d * 16, 16)]
                m_v[jj] = v
                return carry

            lax.fori_loop(0, 16, b_body, 0)
            score_vec = plsc.load_gather(
                m_v, [lane, jnp.zeros((16,), jnp.int32)])
            for i in range(1, 16):
                score_vec = score_vec + plsc.load_gather(
                    m_v, [lane, jnp.full((16,), i, jnp.int32)])
            score_vec = (score_vec + ub_v[pl.ds(g * 16, 16)]
                         + rb_v[pl.ds(g * 16, 16)])
            out_v[pl.ds(c * CH + g * 16, 16)] = score_vec
        return 0

    lax.fori_loop(0, N_CHUNKS, chunk_body, 0)
    pltpu.sync_copy(out_v, out_h.at[pl.ds(wid * B_PER_W, B_PER_W)])


@jax.jit
def _run(ing_flat, user, recipe, ing_table, user_table, user_bias,
         recipe_bias):
    mesh = plsc.VectorSubcoreMesh(core_axis_name="c", subcore_axis_name="s")
    return pl.kernel(
        _sc_body,
        out_type=jax.ShapeDtypeStruct((B,), jnp.float32),
        mesh=mesh,
        compiler_params=pltpu.CompilerParams(
            needs_layout_passes=False, use_tc_tiling_on_sc=False),
        scratch_types=[
            pltpu.VMEM((CH * L,), jnp.int32),
            pltpu.VMEM((CH * L, D), jnp.float32),
            pltpu.VMEM((CH,), jnp.int32),
            pltpu.VMEM((CH,), jnp.int32),
            pltpu.VMEM((CH, D), jnp.float32),
            pltpu.VMEM((CH,), jnp.float32),
            pltpu.VMEM((CH,), jnp.float32),
            pltpu.VMEM((B_PER_W,), jnp.float32),
            pltpu.VMEM((16, 16), jnp.float32),
            pltpu.SemaphoreType.DMA,
            pltpu.SemaphoreType.DMA,
        ],
    )(ing_flat, user, recipe, ing_table, user_table, user_bias, recipe_bias)


def kernel(ingredients, user, recipe, user_table, ing_table, user_bias,
           recipe_bias):
    ing_flat = ingredients.reshape(-1)
    return _run(ing_flat, user, recipe, ing_table, user_table,
                user_bias.reshape(-1), recipe_bias.reshape(-1))


# speedup vs baseline: 2.1636x; 1.0607x over previous
import jax
import jax.numpy as jnp
from jax import lax
from jax.experimental import pallas as pl
from jax.experimental.pallas import tpu as pltpu
from jax.experimental.pallas import tpu_sc as plsc

B = 16384
L = 20
D = 64

NC = 2
NS = 16
NW = NC * NS
B_PER_W = B // NW
CH = 32
N_CHUNKS = B_PER_W // CH
IDX_ROWS = CH * L // 128


def _sc_body(ing_flat, user_h, recipe_h, ing_table, user_table, user_bias,
             recipe_bias, out_h,
             idx0, idx1, uidx0, uidx1, ridx0, ridx1,
             rows0, rows1, urows0, urows1, ub0, ub1, rb0, rb1,
             out_v, m_v, semg0, semg1, semi0, semi1):
    wid = lax.axis_index("s") * NC + lax.axis_index("c")
    lane = lax.iota(jnp.int32, 16)

    idxb = [idx0, idx1]
    uidxb = [uidx0, uidx1]
    ridxb = [ridx0, ridx1]
    rowsb = [rows0, rows1]
    urowsb = [urows0, urows1]
    ubb = [ub0, ub1]
    rbb = [rb0, rb1]
    semg = [semg0, semg1]
    semi = [semi0, semi1]

    def fire_idx(c, p):
        base = wid * B_PER_W + c * CH
        pltpu.async_copy(ing_flat.at[pl.ds(base * L, CH * L)], idxb[p],
                         semi[p])
        pltpu.async_copy(user_h.at[pl.ds(base, CH)], uidxb[p], semi[p])
        pltpu.async_copy(recipe_h.at[pl.ds(base, CH)], ridxb[p], semi[p])

    def wait_idx(p):
        pltpu.make_async_copy(ing_flat.at[pl.ds(0, CH * L)], idxb[p],
                              semi[p]).wait()
        pltpu.make_async_copy(user_h.at[pl.ds(0, CH)], uidxb[p],
                              semi[p]).wait()
        pltpu.make_async_copy(recipe_h.at[pl.ds(0, CH)], ridxb[p],
                              semi[p]).wait()

    def fire_gathers(p):
        for j in range(IDX_ROWS):
            pltpu.async_copy(
                ing_table.at[idxb[p].at[pl.ds(j * 128, 128)]],
                rowsb[p].at[pl.ds(j * 128, 128)], semg[p])
        pltpu.async_copy(user_table.at[uidxb[p]], urowsb[p], semg[p])
        pltpu.async_copy(user_bias.at[uidxb[p]], ubb[p], semg[p])
        pltpu.async_copy(recipe_bias.at[ridxb[p]], rbb[p], semg[p])

    def wait_gathers(p):
        for j in range(IDX_ROWS):
            pltpu.make_async_copy(
                ing_table.at[pl.ds(0, 128)],
                rowsb[p].at[pl.ds(j * 128, 128)], semg[p]).wait()
        pltpu.make_async_copy(user_table.at[pl.ds(0, CH)], urowsb[p],
                              semg[p]).wait()
        pltpu.make_async_copy(user_bias.at[pl.ds(0, CH)], ubb[p],
                              semg[p]).wait()
        pltpu.make_async_copy(recipe_bias.at[pl.ds(0, CH)], rbb[p],
                              semg[p]).wait()

    def compute(c, p):
        rows_v = rowsb[p]
        urows_v = urowsb[p]
        for g in range(CH // 16):
            def b_body(jj, carry, g=g):
                b = g * 16 + jj
                r = b * L
                v = jnp.zeros((16,), jnp.float32)
                for d in range(D // 16):
                    s = rows_v[r, pl.ds(d * 16, 16)]
                    for l in range(1, L):
                        s = s + rows_v[r + l, pl.ds(d * 16, 16)]
                    v = v + s * urows_v[b, pl.ds(d * 16, 16)]
                m_v[jj] = v
                return carry

            lax.fori_loop(0, 16, b_body, 0)
            score_vec = plsc.load_gather(
                m_v, [lane, jnp.zeros((16,), jnp.int32)])
            for i in range(1, 16):
                score_vec = score_vec + plsc.load_gather(
                    m_v, [lane, jnp.full((16,), i, jnp.int32)])
            score_vec = (score_vec + ubb[p][pl.ds(g * 16, 16)]
                         + rbb[p][pl.ds(g * 16, 16)])
            out_v[pl.ds(c * CH + g * 16, 16)] = score_vec

    fire_idx(0, 0)
    fire_idx(1, 1)
    wait_idx(0)
    fire_gathers(0)

    def body(i, _):
        c0 = 2 * i
        wait_gathers(0)
        wait_idx(1)
        fire_gathers(1)
        fire_idx(c0 + 2, 0)
        compute(c0, 0)

        wait_gathers(1)
        wait_idx(0)
        fire_gathers(0)
        fire_idx(c0 + 3, 1)
        compute(c0 + 1, 1)
        return 0

    lax.fori_loop(0, (N_CHUNKS - 2) // 2, body, 0)

    wait_gathers(0)
    wait_idx(1)
    fire_gathers(1)
    compute(N_CHUNKS - 2, 0)
    wait_gathers(1)
    compute(N_CHUNKS - 1, 1)

    pltpu.sync_copy(out_v, out_h.at[pl.ds(wid * B_PER_W, B_PER_W)])


@jax.jit
def _run(ing_flat, user, recipe, ing_table, user_table, user_bias,
         recipe_bias):
    mesh = plsc.VectorSubcoreMesh(core_axis_name="c", subcore_axis_name="s")
    return pl.kernel(
        _sc_body,
        out_type=jax.ShapeDtypeStruct((B,), jnp.float32),
        mesh=mesh,
        compiler_params=pltpu.CompilerParams(
            needs_layout_passes=False, use_tc_tiling_on_sc=False),
        scratch_types=[
            pltpu.VMEM((CH * L,), jnp.int32),
            pltpu.VMEM((CH * L,), jnp.int32),
            pltpu.VMEM((CH,), jnp.int32),
            pltpu.VMEM((CH,), jnp.int32),
            pltpu.VMEM((CH,), jnp.int32),
            pltpu.VMEM((CH,), jnp.int32),
            pltpu.VMEM((CH * L, D), jnp.float32),
            pltpu.VMEM((CH * L, D), jnp.float32),
            pltpu.VMEM((CH, D), jnp.float32),
            pltpu.VMEM((CH, D), jnp.float32),
            pltpu.VMEM((CH,), jnp.float32),
            pltpu.VMEM((CH,), jnp.float32),
            pltpu.VMEM((CH,), jnp.float32),
            pltpu.VMEM((CH,), jnp.float32),
            pltpu.VMEM((B_PER_W,), jnp.float32),
            pltpu.VMEM((16, 16), jnp.float32),
            pltpu.SemaphoreType.DMA,
            pltpu.SemaphoreType.DMA,
            pltpu.SemaphoreType.DMA,
            pltpu.SemaphoreType.DMA,
        ],
    )(ing_flat, user, recipe, ing_table, user_table, user_bias, recipe_bias)


def kernel(ingredients, user, recipe, user_table, ing_table, user_bias,
           recipe_bias):
    ing_flat = ingredients.reshape(-1)
    return _run(ing_flat, user, recipe, ing_table, user_table,
                user_bias.reshape(-1), recipe_bias.reshape(-1))
